# baseline (speedup 1.0000x reference)
import functools

import jax
import jax.numpy as jnp
from jax import lax
from jax.experimental import pallas as pl
from jax.experimental.pallas import tpu as pltpu
from jax.experimental.pallas import tpu_sc as plsc

N = 10000
E = 320000
D_IN = 128
D_E = 16
H = 4
OUT_N = 32
OUT_E = 16
FE = H * OUT_E
FN = H * OUT_N

ACC_W = 144
NW = 32
B = 40
EPW = E // NW
NCH = EPW // B
N_PAD = 10112
RPT = N_PAD // 16

_mesh = plsc.VectorSubcoreMesh(core_axis_name="c", subcore_axis_name="s")


@functools.partial(
    pl.kernel,
    out_type=(
        jax.ShapeDtypeStruct((E, FE), jnp.float32),
        jax.ShapeDtypeStruct((2, N_PAD, ACC_W), jnp.float32),
    ),
    mesh=_mesh,
    compiler_params=pltpu.CompilerParams(use_tc_tiling_on_sc=False),
    scratch_types=[
        pltpu.VMEM((B,), jnp.int32),
        pltpu.VMEM((B,), jnp.int32),
        pltpu.VMEM((B, 192), jnp.float32),
        pltpu.VMEM((B, 64), jnp.float32),
        pltpu.VMEM((B, 64), jnp.float32),
        pltpu.VMEM((B, 64), jnp.float32),
        pltpu.VMEM((B, ACC_W), jnp.float32),
        pltpu.VMEM((16,), jnp.float32),
        pltpu.VMEM_SHARED((N_PAD, ACC_W), jnp.float32),
        pltpu.SemaphoreType.DMA,
    ],
)
def _edge_kernel(src_hbm, dst_hbm, a_hbm, bt_hbm, pe_hbm, wsum_hbm, zeros_hbm,
                 f_hbm, part_hbm,
                 src_v, dst_v, gs_v, gd_v, pe_v, f_v, w_v, wsum_v, acc_sh, sem):
    cid = lax.axis_index("c")
    sid = lax.axis_index("s")
    wid = sid * 2 + cid

    pltpu.sync_copy(wsum_hbm, wsum_v)
    pltpu.sync_copy(zeros_hbm.at[pl.ds(sid * RPT, RPT)],
                    acc_sh.at[pl.ds(sid * RPT, RPT)])
    plsc.subcore_barrier()

    wsum = wsum_v[...]
    iota = lax.iota(jnp.int32, 16)
    perms = [lax.bitwise_xor(iota, k) for k in (8, 4, 2, 1)]

    def lanesum(x):
        for p in perms:
            x = x + x.at[p].get(mode="promise_in_bounds")
        return x

    def chunk_body(g, carry):
        base = pl.multiple_of(wid * EPW + g * B, B)
        pltpu.sync_copy(src_hbm.at[pl.ds(base, B)], src_v)
        pltpu.sync_copy(dst_hbm.at[pl.ds(base, B)], dst_v)
        pltpu.async_copy(a_hbm.at[src_v], gs_v, sem).wait()
        pltpu.async_copy(bt_hbm.at[dst_v], gd_v, sem).wait()
        pltpu.sync_copy(pe_hbm.at[pl.ds(base, B)], pe_v)

        def edge_body(b, carry2):
            fs = []
            for h in range(H):
                sl = pl.ds(h * 16, 16)
                fh = gs_v[b, sl] + gd_v[b, sl] + pe_v[b, sl]
                fs.append(jnp.maximum(fh, fh * 0.01))
            for h in range(H):
                f_v[b, pl.ds(h * 16, 16)] = fs[h]
            ts = [fh * wsum for fh in fs]
            for p in perms:
                ts = [t + t.at[p].get(mode="promise_in_bounds") for t in ts]
            ss = [jnp.exp(t) for t in ts]
            svec = jnp.zeros((16,), jnp.float32)
            for h in range(H):
                svec = jnp.where(iota == h, ss[h], svec)
                hn0 = gs_v[b, pl.ds(64 + h * 32, 16)]
                hn1 = gs_v[b, pl.ds(64 + h * 32 + 16, 16)]
                w_v[b, pl.ds(h * 32, 16)] = ss[h] * hn0
                w_v[b, pl.ds(h * 32 + 16, 16)] = ss[h] * hn1
            w_v[b, pl.ds(128, 16)] = svec
            return carry2

        lax.fori_loop(0, B, edge_body, 0, unroll=2)
        pltpu.sync_copy(f_v, f_hbm.at[pl.ds(base, B)])
        pltpu.sync_copy(w_v, acc_sh.at[dst_v], add=True)
        return carry

    lax.fori_loop(0, NCH, chunk_body, 0)
    plsc.subcore_barrier()
    pltpu.sync_copy(acc_sh.at[pl.ds(sid * RPT, RPT)],
                    part_hbm.at[cid, pl.ds(sid * RPT, RPT)])


def _node_tables_body(x_ref, wt_ref, b_ref, a_ref, bt_ref):
    z = jnp.dot(x_ref[...], wt_ref[...], preferred_element_type=jnp.float32)
    z = z + b_ref[...]
    a_ref[...] = z[:, :192]
    bt_ref[...] = z[:, 192:]


def _pe_body(x_ref, wt_ref, b_ref, o_ref):
    o_ref[...] = jnp.dot(x_ref[...], wt_ref[...],
                         preferred_element_type=jnp.float32) + b_ref[...]


def _combine_body(p_ref, s_ref, o_ref):
    p = p_ref[...]
    u = p[0, :, :FN] + p[1, :, :FN]
    d = p[0, :, FN:FN + H] + p[1, :, FN:FN + H]
    db = jnp.dot(d, s_ref[...], preferred_element_type=jnp.float32)
    o_ref[...] = u / jnp.maximum(db, 1e-9)


def kernel(nfeats, efeats, edge_index, W_nodes, b_nodes, W_edges, b_edges,
           W_attn):
    src = edge_index[0]
    dst = edge_index[1]
    W1 = W_edges[:, :D_IN]
    W2 = W_edges[:, D_IN:D_IN + D_E]
    W3 = W_edges[:, D_IN + D_E:]
    wsum = jnp.sum(W_attn, axis=0)

    Wbig_t = jnp.concatenate([W1, W_nodes, W3], axis=0).T
    bbig = jnp.concatenate(
        [jnp.zeros((FE,), jnp.float32), b_nodes,
         jnp.zeros((FE,), jnp.float32)]).reshape(1, 256)
    BN = 1000
    A, Bt = pl.pallas_call(
        _node_tables_body,
        grid=(N // BN,),
        in_specs=[
            pl.BlockSpec((BN, D_IN), lambda i: (i, 0)),
            pl.BlockSpec((D_IN, 256), lambda i: (0, 0)),
            pl.BlockSpec((1, 256), lambda i: (0, 0)),
        ],
        out_specs=[
            pl.BlockSpec((BN, 192), lambda i: (i, 0)),
            pl.BlockSpec((BN, 64), lambda i: (i, 0)),
        ],
        out_shape=[
            jax.ShapeDtypeStruct((N, 192), jnp.float32),
            jax.ShapeDtypeStruct((N, 64), jnp.float32),
        ],
    )(nfeats, Wbig_t, bbig)

    BE = 4000
    pe = pl.pallas_call(
        _pe_body,
        grid=(E // BE,),
        in_specs=[
            pl.BlockSpec((BE, D_E), lambda i: (i, 0)),
            pl.BlockSpec((D_E, FE), lambda i: (0, 0)),
            pl.BlockSpec((1, FE), lambda i: (0, 0)),
        ],
        out_specs=pl.BlockSpec((BE, FE), lambda i: (i, 0)),
        out_shape=jax.ShapeDtypeStruct((E, FE), jnp.float32),
    )(efeats, W2.T, b_edges.reshape(1, FE))

    zeros = jnp.zeros((N_PAD, ACC_W), jnp.float32)
    f_out, partials = _edge_kernel(src, dst, A, Bt, pe, wsum, zeros)

    sel = jnp.repeat(jnp.eye(H, dtype=jnp.float32), OUT_N, axis=1)
    BC = 632
    nfeats_out = pl.pallas_call(
        _combine_body,
        grid=(N_PAD // BC,),
        in_specs=[
            pl.BlockSpec((2, BC, ACC_W), lambda i: (0, i, 0)),
            pl.BlockSpec((H, FN), lambda i: (0, 0)),
        ],
        out_specs=pl.BlockSpec((BC, FN), lambda i: (i, 0)),
        out_shape=jax.ShapeDtypeStruct((N, FN), jnp.float32),
    )(partials, sel)

    return (nfeats_out, f_out)


# speedup vs baseline: 1.2444x; 1.2444x over previous
import functools

import jax
import jax.numpy as jnp
from jax import lax
from jax.experimental import pallas as pl
from jax.experimental.pallas import tpu as pltpu
from jax.experimental.pallas import tpu_sc as plsc

N = 10000
E = 320000
D_IN = 128
D_E = 16
H = 4
OUT_N = 32
OUT_E = 16
FE = H * OUT_E
FN = H * OUT_N

ACC_W = 144
NW = 32
B = 40
EPW = E // NW
NCH = EPW // B
N_PAD = 10112
RPT = N_PAD // 16

_mesh = plsc.VectorSubcoreMesh(core_axis_name="c", subcore_axis_name="s")


@functools.partial(
    pl.kernel,
    out_type=(
        jax.ShapeDtypeStruct((E, FE), jnp.float32),
        jax.ShapeDtypeStruct((2, N_PAD, ACC_W), jnp.float32),
    ),
    mesh=_mesh,
    compiler_params=pltpu.CompilerParams(use_tc_tiling_on_sc=False),
    scratch_types=[
        pltpu.VMEM((B,), jnp.int32),
        pltpu.VMEM((B,), jnp.int32),
        pltpu.VMEM((B, 192), jnp.float32),
        pltpu.VMEM((B, 64), jnp.float32),
        pltpu.VMEM((B, 64), jnp.float32),
        pltpu.VMEM((B,), jnp.int32),
        pltpu.VMEM((B,), jnp.int32),
        pltpu.VMEM((B, 192), jnp.float32),
        pltpu.VMEM((B, 64), jnp.float32),
        pltpu.VMEM((B, 64), jnp.float32),
        pltpu.VMEM((B, 64), jnp.float32),
        pltpu.VMEM((B, ACC_W), jnp.float32),
        pltpu.VMEM((16,), jnp.float32),
        pltpu.VMEM_SHARED((N_PAD, ACC_W), jnp.float32),
        pltpu.SemaphoreType.DMA,
        pltpu.SemaphoreType.DMA,
    ],
)
def _edge_kernel(src_hbm, dst_hbm, a_hbm, bt_hbm, pe_hbm, wsum_hbm, zeros_hbm,
                 f_hbm, part_hbm,
                 src_v, dst_v, gs_v, gd_v, pe_v,
                 src_w, dst_w, gs_w, gd_w, pe_w,
                 f_v, w_v, wsum_v, acc_sh, sem, sem2):
    cid = lax.axis_index("c")
    sid = lax.axis_index("s")
    wid = sid * 2 + cid
    buf0 = (src_v, dst_v, gs_v, gd_v, pe_v, sem)
    buf1 = (src_w, dst_w, gs_w, gd_w, pe_w, sem2)

    pltpu.sync_copy(wsum_hbm, wsum_v)
    pltpu.sync_copy(zeros_hbm.at[pl.ds(sid * RPT, RPT)],
                    acc_sh.at[pl.ds(sid * RPT, RPT)])
    plsc.subcore_barrier()

    wsum = wsum_v[...]
    iota = lax.iota(jnp.int32, 16)
    perms = [lax.bitwise_xor(iota, k) for k in (8, 4, 2, 1)]

    def lanesum(x):
        for p in perms:
            x = x + x.at[p].get(mode="promise_in_bounds")
        return x

    def issue(buf, g):
        srcb, dstb, gsb, gdb, peb, semg = buf
        base = pl.multiple_of(wid * EPW + g * B, B)
        pltpu.sync_copy(src_hbm.at[pl.ds(base, B)], srcb)
        pltpu.sync_copy(dst_hbm.at[pl.ds(base, B)], dstb)
        pltpu.async_copy(a_hbm.at[srcb], gsb, semg)
        pltpu.async_copy(bt_hbm.at[dstb], gdb, semg)
        pltpu.sync_copy(pe_hbm.at[pl.ds(base, B)], peb)

    def wait_gathers(buf):
        srcb, dstb, gsb, gdb, peb, semg = buf
        pltpu.make_async_copy(a_hbm.at[srcb], gsb, semg).wait()
        pltpu.make_async_copy(bt_hbm.at[dstb], gdb, semg).wait()

    def compute(buf, g):
        srcb, dstb, gs_v, gd_v, pe_v, semg = buf
        base = pl.multiple_of(wid * EPW + g * B, B)

        def edge_body(b, carry2):
            fs = []
            for h in range(H):
                sl = pl.ds(h * 16, 16)
                fh = gs_v[b, sl] + gd_v[b, sl] + pe_v[b, sl]
                fs.append(jnp.maximum(fh, fh * 0.01))
            for h in range(H):
                f_v[b, pl.ds(h * 16, 16)] = fs[h]
            ts = [fh * wsum for fh in fs]
            for p in perms:
                ts = [t + t.at[p].get(mode="promise_in_bounds") for t in ts]
            ss = [jnp.exp(t) for t in ts]
            svec = jnp.zeros((16,), jnp.float32)
            for h in range(H):
                svec = jnp.where(iota == h, ss[h], svec)
                hn0 = gs_v[b, pl.ds(64 + h * 32, 16)]
                hn1 = gs_v[b, pl.ds(64 + h * 32 + 16, 16)]
                w_v[b, pl.ds(h * 32, 16)] = ss[h] * hn0
                w_v[b, pl.ds(h * 32 + 16, 16)] = ss[h] * hn1
            w_v[b, pl.ds(128, 16)] = svec
            return carry2

        lax.fori_loop(0, B, edge_body, 0, unroll=2)
        pltpu.sync_copy(f_v, f_hbm.at[pl.ds(base, B)])
        pltpu.sync_copy(w_v, acc_sh.at[dstb], add=True)

    issue(buf0, 0)

    def pair_body(q, carry):
        geven = q * 2
        issue(buf1, geven + 1)
        wait_gathers(buf0)
        compute(buf0, geven)

        @pl.when(q < NCH // 2 - 1)
        def _prefetch():
            issue(buf0, geven + 2)

        wait_gathers(buf1)
        compute(buf1, geven + 1)
        return carry

    lax.fori_loop(0, NCH // 2, pair_body, 0)
    plsc.subcore_barrier()
    pltpu.sync_copy(acc_sh.at[pl.ds(sid * RPT, RPT)],
                    part_hbm.at[cid, pl.ds(sid * RPT, RPT)])


def _node_tables_body(x_ref, wt_ref, b_ref, a_ref, bt_ref):
    z = jnp.dot(x_ref[...], wt_ref[...], preferred_element_type=jnp.float32)
    z = z + b_ref[...]
    a_ref[...] = z[:, :192]
    bt_ref[...] = z[:, 192:]


def _pe_body(x_ref, wt_ref, b_ref, o_ref):
    o_ref[...] = jnp.dot(x_ref[...], wt_ref[...],
                         preferred_element_type=jnp.float32) + b_ref[...]


def _combine_body(p_ref, s_ref, o_ref):
    p = p_ref[...]
    u = p[0, :, :FN] + p[1, :, :FN]
    d = p[0, :, FN:FN + H] + p[1, :, FN:FN + H]
    db = jnp.dot(d, s_ref[...], preferred_element_type=jnp.float32)
    o_ref[...] = u / jnp.maximum(db, 1e-9)


def kernel(nfeats, efeats, edge_index, W_nodes, b_nodes, W_edges, b_edges,
           W_attn):
    src = edge_index[0]
    dst = edge_index[1]
    W1 = W_edges[:, :D_IN]
    W2 = W_edges[:, D_IN:D_IN + D_E]
    W3 = W_edges[:, D_IN + D_E:]
    wsum = jnp.sum(W_attn, axis=0)

    Wbig_t = jnp.concatenate([W1, W_nodes, W3], axis=0).T
    bbig = jnp.concatenate(
        [jnp.zeros((FE,), jnp.float32), b_nodes,
         jnp.zeros((FE,), jnp.float32)]).reshape(1, 256)
    BN = 1000
    A, Bt = pl.pallas_call(
        _node_tables_body,
        grid=(N // BN,),
        in_specs=[
            pl.BlockSpec((BN, D_IN), lambda i: (i, 0)),
            pl.BlockSpec((D_IN, 256), lambda i: (0, 0)),
            pl.BlockSpec((1, 256), lambda i: (0, 0)),
        ],
        out_specs=[
            pl.BlockSpec((BN, 192), lambda i: (i, 0)),
            pl.BlockSpec((BN, 64), lambda i: (i, 0)),
        ],
        out_shape=[
            jax.ShapeDtypeStruct((N, 192), jnp.float32),
            jax.ShapeDtypeStruct((N, 64), jnp.float32),
        ],
    )(nfeats, Wbig_t, bbig)

    BE = 4000
    pe = pl.pallas_call(
        _pe_body,
        grid=(E // BE,),
        in_specs=[
            pl.BlockSpec((BE, D_E), lambda i: (i, 0)),
            pl.BlockSpec((D_E, FE), lambda i: (0, 0)),
            pl.BlockSpec((1, FE), lambda i: (0, 0)),
        ],
        out_specs=pl.BlockSpec((BE, FE), lambda i: (i, 0)),
        out_shape=jax.ShapeDtypeStruct((E, FE), jnp.float32),
    )(efeats, W2.T, b_edges.reshape(1, FE))

    zeros = jnp.zeros((N_PAD, ACC_W), jnp.float32)
    f_out, partials = _edge_kernel(src, dst, A, Bt, pe, wsum, zeros)

    sel = jnp.repeat(jnp.eye(H, dtype=jnp.float32), OUT_N, axis=1)
    BC = 632
    nfeats_out = pl.pallas_call(
        _combine_body,
        grid=(N_PAD // BC,),
        in_specs=[
            pl.BlockSpec((2, BC, ACC_W), lambda i: (0, i, 0)),
            pl.BlockSpec((H, FN), lambda i: (0, 0)),
        ],
        out_specs=pl.BlockSpec((BC, FN), lambda i: (i, 0)),
        out_shape=jax.ShapeDtypeStruct((N, FN), jnp.float32),
    )(partials, sel)

    return (nfeats_out, f_out)
